# baseline (speedup 1.0000x reference)
import functools
import math

import jax
import jax.numpy as jnp
from jax import lax
from jax.experimental import pallas as pl
from jax.experimental.pallas import tpu as pltpu
from jax.experimental.pallas import tpu_sc as plsc

_ROUND = 32


@functools.lru_cache(maxsize=None)
def _make_gather(B, NT, TR, D, num_objs):
    info = plsc.get_sparse_core_info()
    NC, NS, L = info.num_cores, info.num_subcores, info.num_lanes
    NW = NC * NS
    assert B % (8 * NW) == 0 and D % L == 0
    b_per_w = B // NW
    n_rounds = b_per_w // _ROUND
    n_pairs = n_rounds // 2

    mesh = plsc.VectorSubcoreMesh(core_axis_name="c", subcore_axis_name="s")

    @functools.partial(
        pl.kernel,
        mesh=mesh,
        out_type=jax.ShapeDtypeStruct((B, D), jnp.float32),
        compiler_params=pltpu.CompilerParams(use_tc_tiling_on_sc=True),
        scratch_types=[
            pltpu.VMEM((b_per_w,), jnp.int32),
            pltpu.VMEM((b_per_w,), jnp.int32),
            pltpu.VMEM((_ROUND, TR, D), jnp.float32),
            pltpu.VMEM((_ROUND, TR, D), jnp.float32),
            pltpu.VMEM((2, _ROUND, D), jnp.float32),
            pltpu.SemaphoreType.DMA,
            pltpu.SemaphoreType.DMA,
            pltpu.SemaphoreType.DMA,
        ],
    )
    def gather_kernel(
        tid_hbm, sub_hbm, w3_hbm, out_hbm,
        tid_v, sub_v, tiles0_v, tiles1_v, stage_v, gsem0, gsem1, osem,
    ):
        wid = lax.axis_index("s") * NC + lax.axis_index("c")
        base = wid * b_per_w
        pltpu.sync_copy(tid_hbm.at[pl.ds(base, b_per_w)], tid_v)
        pltpu.sync_copy(sub_hbm.at[pl.ds(base, b_per_w)], sub_v)

        def fire(k, tiles, sem):
            def fire_g(g, _2):
                tvec = tid_v[pl.ds(k * _ROUND + g * L, L)]
                for j in range(L):
                    pltpu.async_copy(
                        w3_hbm.at[tvec[j]], tiles.at[g * L + j], sem
                    )
                return _2

            lax.fori_loop(0, _ROUND // L, fire_g, None)

        def drain_tiles(sem):
            pltpu.make_async_copy(
                w3_hbm.at[pl.ds(0, _ROUND)], tiles0_v, sem
            ).wait()

        def reclaim_stage(sl):
            pltpu.make_async_copy(
                out_hbm.at[pl.ds(0, _ROUND), :], stage_v.at[sl], osem
            ).wait()

        def extract_and_put(k, sl, tiles):
            def ext_g(g, _2):
                svec = sub_v[pl.ds(k * _ROUND + g * L, L)]
                for j in range(L):
                    slot = g * L + j
                    s = svec[j]
                    for c in range(D // L):
                        stage_v[sl, slot, pl.ds(c * L, L)] = (
                            tiles[slot, s, pl.ds(c * L, L)]
                        )
                return _2

            lax.fori_loop(0, _ROUND // L, ext_g, None)
            pltpu.async_copy(
                stage_v.at[sl],
                out_hbm.at[pl.ds(base + k * _ROUND, _ROUND), :],
                osem,
            )

        fire(0, tiles0_v, gsem0)

        def pair_body(kk, _):
            k0 = 2 * kk
            fire(k0 + 1, tiles1_v, gsem1)
            drain_tiles(gsem0)

            @pl.when(kk >= 1)
            def _r0():
                reclaim_stage(0)

            extract_and_put(k0, 0, tiles0_v)

            @pl.when(kk + 1 < n_pairs)
            def _f0():
                fire(k0 + 2, tiles0_v, gsem0)

            drain_tiles(gsem1)

            @pl.when(kk >= 1)
            def _r1():
                reclaim_stage(1)

            extract_and_put(k0 + 1, 1, tiles1_v)
            return _

        lax.fori_loop(0, n_pairs, pair_body, None)
        reclaim_stage(0)
        reclaim_stage(1)

    return gather_kernel


def kernel(labels, W):
    B = labels.shape[0]
    V, D = W.shape
    num_objs = math.isqrt(V)
    w3 = W.reshape(V // 8, 8, D)
    r = labels[:, 0].astype(jnp.int32) * num_objs + labels[:, 1].astype(jnp.int32)
    return _make_gather(B, V // 8, 8, D, num_objs)(r >> 3, r & 7, w3)
